# baseline (speedup 1.0000x reference)
import functools

import jax
import jax.numpy as jnp
from jax import lax
from jax.experimental import pallas as pl
from jax.experimental.pallas import tpu as pltpu
from jax.experimental.pallas import tpu_sc as plsc

N = 10000
C = 128
K = 16
NPAD = 10240
NEG_SLOPE = 0.2


KNN_RB = 256
KNN_T = 2 * K


BIGF = 3.0e38


def _knn_body(xb_ref, xf_ref, sqb_ref, sqr_ref, out_ref, d_ref, iota_ref,
              m_ref, acc_ref):
    t = pl.program_id(1)

    @pl.when(t == 0)
    def _init():
        xb = xb_ref[...]
        xf = xf_ref[...]
        d2 = lax.dot_general(xb, xf, (((1,), (1,)), ((), ())),
                             preferred_element_type=jnp.float32)
        d = sqb_ref[...] - 2.0 * d2 + sqr_ref[...]
        rb = pl.program_id(0)
        cols = lax.broadcasted_iota(jnp.int32, (KNN_RB, NPAD), 1)
        rows = lax.broadcasted_iota(jnp.int32, (KNN_RB, NPAD), 0) + rb * KNN_RB
        d = jnp.where((cols == rows) | (cols >= N), BIGF, d)
        d_ref[...] = d
        iota_ref[...] = cols.astype(jnp.float32)
        m_ref[...] = jnp.min(d, axis=1, keepdims=True)
        acc_ref[...] = jnp.zeros((KNN_RB, KNN_T), jnp.int32)

    @pl.when(t > 0)
    def _extract():
        d = d_ref[...]
        iotaf = iota_ref[...]
        m = m_ref[...]
        cand = jnp.where(d == m, iotaf, BIGF)
        jf = jnp.min(cand, axis=1, keepdims=True)
        lane = lax.broadcasted_iota(jnp.int32, (KNN_RB, KNN_T), 1)
        acc_ref[...] = jnp.where(lane == t - 1, jf.astype(jnp.int32),
                                 acc_ref[...])

        @pl.when(t < KNN_T)
        def _mask_and_next_min():
            dn = jnp.where(iotaf == jf, BIGF, d)
            d_ref[...] = dn
            m_ref[...] = jnp.min(dn, axis=1, keepdims=True)

        @pl.when(t == KNN_T)
        def _emit():
            out_ref[...] = acc_ref[...]


def knn_top32(x_pad):
    sq = jnp.sum(x_pad * x_pad, axis=1)
    grid = (NPAD // KNN_RB, KNN_T + 1)
    return pl.pallas_call(
        _knn_body,
        grid=grid,
        in_specs=[
            pl.BlockSpec((KNN_RB, C), lambda rb, t: (rb, 0)),
            pl.BlockSpec((NPAD, C), lambda rb, t: (0, 0)),
            pl.BlockSpec((KNN_RB, 1), lambda rb, t: (rb, 0)),
            pl.BlockSpec((1, NPAD), lambda rb, t: (0, 0)),
        ],
        out_specs=pl.BlockSpec((KNN_RB, KNN_T), lambda rb, t: (rb, 0)),
        out_shape=jax.ShapeDtypeStruct((NPAD, KNN_T), jnp.int32),
        scratch_shapes=[
            pltpu.VMEM((KNN_RB, NPAD), jnp.float32),
            pltpu.VMEM((KNN_RB, NPAD), jnp.float32),
            pltpu.VMEM((KNN_RB, 1), jnp.float32),
            pltpu.VMEM((KNN_RB, KNN_T), jnp.int32),
        ],
    )(x_pad, x_pad, sq[:, None], sq[None, :])


MM_RB = 1280


def _mm_body(x_ref, w_ref, b_ref, out_ref):
    acc = jnp.dot(x_ref[...], w_ref[...], preferred_element_type=jnp.float32,
                  precision=lax.Precision.HIGHEST)
    out_ref[...] = acc + b_ref[...]


def node_matmul(xc, wcat, bcat):
    n, in_ch = xc.shape
    oc = wcat.shape[1]
    return pl.pallas_call(
        _mm_body,
        grid=(n // MM_RB,),
        in_specs=[
            pl.BlockSpec((MM_RB, in_ch), lambda i: (i, 0)),
            pl.BlockSpec((in_ch, oc), lambda i: (0, 0)),
            pl.BlockSpec((1, oc), lambda i: (0, 0)),
        ],
        out_specs=pl.BlockSpec((MM_RB, oc), lambda i: (i, 0)),
        out_shape=jax.ShapeDtypeStruct((n, oc), jnp.float32),
    )(xc, wcat, bcat)


SC_INFO = plsc.get_sparse_core_info()
SC_NC = SC_INFO.num_cores
SC_NS = SC_INFO.num_subcores
SC_NW = SC_NC * SC_NS
SC_NPW = NPAD // SC_NW
SC_CH = 8
SC_NCHUNK = SC_NPW // SC_CH
SC_L = 16
SC_G = C // SC_L


def _sc_gather_max(a_hbm, bp_hbm, idx_hbm, out_hbm, idx_v, rows_v, a_v, h_v,
                   sem):
    wid = lax.axis_index("s") * SC_NC + lax.axis_index("c")
    node0 = wid * SC_NPW

    def chunk_body(ci, carry):
        base = node0 + ci * SC_CH
        pltpu.sync_copy(idx_hbm.at[pl.ds(base * K, SC_CH * K)], idx_v)
        pltpu.async_copy(bp_hbm.at[idx_v], rows_v, sem).wait()
        pltpu.sync_copy(a_hbm.at[pl.ds(base, SC_CH)], a_v)

        def node_body(n, carry2):
            for g in range(SC_G):
                z = a_v[n, pl.ds(g * SC_L, SC_L)]
                acc = None
                for j in range(K):
                    m = z + rows_v[n * K + j, pl.ds(g * SC_L, SC_L)]
                    m = jnp.maximum(m, NEG_SLOPE * m)
                    acc = m if acc is None else jnp.maximum(acc, m)
                h_v[n, pl.ds(g * SC_L, SC_L)] = acc
            return carry2

        lax.fori_loop(0, SC_CH, node_body, 0, unroll=False)
        pltpu.sync_copy(h_v, out_hbm.at[pl.ds(base, SC_CH)])
        return carry

    lax.fori_loop(0, SC_NCHUNK, chunk_body, 0, unroll=False)


def sc_gather_max(a_pad, bp_pad, idx_flat):
    mesh = plsc.VectorSubcoreMesh(core_axis_name="c", subcore_axis_name="s")
    kern = functools.partial(
        pl.kernel,
        mesh=mesh,
        out_type=jax.ShapeDtypeStruct((NPAD, C), jnp.float32),
        scratch_types=[
            pltpu.VMEM((SC_CH * K,), jnp.int32),
            pltpu.VMEM((SC_CH * K, C), jnp.float32),
            pltpu.VMEM((SC_CH, C), jnp.float32),
            pltpu.VMEM((SC_CH, C), jnp.float32),
            pltpu.SemaphoreType.DMA,
        ],
    )(_sc_gather_max)
    return kern(a_pad, bp_pad, idx_flat)


CB_RB = 2560


def _combine_body(x_ref, h00_ref, h01_ref, h10_ref, h11_ref, s0_ref, s1_ref,
                  s2_ref, out_ref):
    x = x_ref[...]
    g = None
    for ha_ref, hb_ref in ((h00_ref, h01_ref), (h10_ref, h11_ref)):
        x3 = jnp.concatenate([x, ha_ref[...], hb_ref[...]], axis=1)
        gb = None
        for s_ref in (s0_ref, s1_ref, s2_ref):
            sel = jnp.dot(x3, s_ref[...], preferred_element_type=jnp.float32,
                          precision=lax.Precision.HIGHEST)
            gb = sel if gb is None else jnp.maximum(gb, sel)
        g = gb if g is None else jnp.maximum(g, gb)
    out_ref[...] = g + x


def combine(x_pad, h00, h01, h10, h11, s0, s1, s2):
    return pl.pallas_call(
        _combine_body,
        grid=(NPAD // CB_RB,),
        in_specs=[pl.BlockSpec((CB_RB, C), lambda i: (i, 0))] * 5
        + [pl.BlockSpec((3 * C, C), lambda i: (0, 0))] * 3,
        out_specs=pl.BlockSpec((CB_RB, C), lambda i: (i, 0)),
        out_shape=jax.ShapeDtypeStruct((NPAD, C), jnp.float32),
    )(x_pad, h00, h01, h10, h11, s0, s1, s2)


def kernel(x, W0_0, b0_0, W0_1, b0_1, W1_0, b1_0, W1_1, b1_1):
    params = [
        [(W0_0, b0_0), (W0_1, b0_1)],
        [(W1_0, b1_0), (W1_1, b1_1)],
    ]
    x_pad = jnp.pad(x, ((0, NPAD - N), (0, 0)))

    idx32 = knn_top32(x_pad)

    hs = [[None, None], [None, None]]
    for b in range(2):
        nbr = idx32[:, :K] if b == 0 else idx32[:, 0 : 2 * K : 2]
        idx_flat = nbr.reshape(-1)
        xc = x_pad
        for blk in range(2):
            W, bias = params[b][blk]
            in_ch = xc.shape[1]
            wcat = jnp.concatenate([W[:in_ch] - W[in_ch:], W[in_ch:]], axis=1)
            bcat = jnp.concatenate([bias, jnp.zeros((C,), jnp.float32)])[None, :]
            ab = node_matmul(xc, wcat, bcat)
            a_pad, bp_pad = ab[:, :C], ab[:, C:]
            h = sc_gather_max(a_pad, bp_pad, idx_flat)
            hs[b][blk] = h
            xc = jnp.concatenate([xc, h], axis=-1)

    r = jnp.arange(3 * C)[:, None]
    c3 = 3 * jnp.arange(C)[None, :]
    sels = [(r == c3 + t).astype(jnp.float32) for t in range(3)]
    out = combine(x_pad, hs[0][0], hs[0][1], hs[1][0], hs[1][1], *sels)
    return out[:N]


# speedup vs baseline: 1.0879x; 1.0879x over previous
import functools

import jax
import jax.numpy as jnp
from jax import lax
from jax.experimental import pallas as pl
from jax.experimental.pallas import tpu as pltpu
from jax.experimental.pallas import tpu_sc as plsc

N = 10000
C = 128
K = 16
NPAD = 10240
NEG_SLOPE = 0.2


KNN_RB = 256
KNN_T = 2 * K


BIGF = 3.0e38


def _knn_body(xb_ref, xf_ref, sqb_ref, sqr_ref, out_ref, d_ref,
              m_ref, acc_ref):
    t = pl.program_id(1)

    @pl.when(t == 0)
    def _init():
        xb = xb_ref[...]
        xf = xf_ref[...]
        d2 = lax.dot_general(xb, xf, (((1,), (1,)), ((), ())),
                             preferred_element_type=jnp.float32)
        d = sqb_ref[...] - 2.0 * d2 + sqr_ref[...]
        rb = pl.program_id(0)
        cols = lax.broadcasted_iota(jnp.int32, (KNN_RB, NPAD), 1)
        rows = lax.broadcasted_iota(jnp.int32, (KNN_RB, NPAD), 0) + rb * KNN_RB
        d = jnp.where((cols == rows) | (cols >= N), BIGF, d)
        d_ref[...] = d
        m_ref[...] = jnp.min(d, axis=1, keepdims=True)
        acc_ref[...] = jnp.zeros((KNN_RB, KNN_T), jnp.int32)

    @pl.when(t > 0)
    def _extract():
        d = d_ref[...]
        m = m_ref[...]
        iota_b = lax.broadcasted_iota(jnp.int32, (KNN_RB, NPAD), 1
                                      ).astype(jnp.float32)
        cand = jnp.where(d == m, iota_b, BIGF)
        jf = jnp.min(cand, axis=1, keepdims=True)
        lane = lax.broadcasted_iota(jnp.int32, (KNN_RB, KNN_T), 1)
        acc_ref[...] = jnp.where(lane == t - 1, jf.astype(jnp.int32),
                                 acc_ref[...])

        @pl.when(t < KNN_T)
        def _mask_and_next_min():
            iota_a = lax.broadcasted_iota(jnp.int32, (KNN_RB, NPAD), 1)
            dn = jnp.where(iota_a == jf.astype(jnp.int32), BIGF, d)
            d_ref[...] = dn
            m_ref[...] = jnp.min(dn, axis=1, keepdims=True)

        @pl.when(t == KNN_T)
        def _emit():
            out_ref[...] = acc_ref[...]


def knn_top32(x_pad):
    sq = jnp.sum(x_pad * x_pad, axis=1)
    grid = (NPAD // KNN_RB, KNN_T + 1)
    return pl.pallas_call(
        _knn_body,
        grid=grid,
        in_specs=[
            pl.BlockSpec((KNN_RB, C), lambda rb, t: (rb, 0)),
            pl.BlockSpec((NPAD, C), lambda rb, t: (0, 0)),
            pl.BlockSpec((KNN_RB, 1), lambda rb, t: (rb, 0)),
            pl.BlockSpec((1, NPAD), lambda rb, t: (0, 0)),
        ],
        out_specs=pl.BlockSpec((KNN_RB, KNN_T), lambda rb, t: (rb, 0)),
        out_shape=jax.ShapeDtypeStruct((NPAD, KNN_T), jnp.int32),
        scratch_shapes=[
            pltpu.VMEM((KNN_RB, NPAD), jnp.float32),
            pltpu.VMEM((KNN_RB, 1), jnp.float32),
            pltpu.VMEM((KNN_RB, KNN_T), jnp.int32),
        ],
    )(x_pad, x_pad, sq[:, None], sq[None, :])


MM_RB = 1280


def _mm_body(x_ref, w_ref, b_ref, out_ref):
    acc = jnp.dot(x_ref[...], w_ref[...], preferred_element_type=jnp.float32,
                  precision=lax.Precision.HIGHEST)
    out_ref[...] = acc + b_ref[...]


def node_matmul(xc, wcat, bcat):
    n, in_ch = xc.shape
    oc = wcat.shape[1]
    return pl.pallas_call(
        _mm_body,
        grid=(n // MM_RB,),
        in_specs=[
            pl.BlockSpec((MM_RB, in_ch), lambda i: (i, 0)),
            pl.BlockSpec((in_ch, oc), lambda i: (0, 0)),
            pl.BlockSpec((1, oc), lambda i: (0, 0)),
        ],
        out_specs=pl.BlockSpec((MM_RB, oc), lambda i: (i, 0)),
        out_shape=jax.ShapeDtypeStruct((n, oc), jnp.float32),
    )(xc, wcat, bcat)


SC_INFO = plsc.get_sparse_core_info()
SC_NC = SC_INFO.num_cores
SC_NS = SC_INFO.num_subcores
SC_NW = SC_NC * SC_NS
SC_NPW = NPAD // SC_NW
SC_CH = 8
SC_NCHUNK = SC_NPW // SC_CH
SC_L = 16
SC_G = C // SC_L


def _sc_gather_max(a_hbm, bp_hbm, idx_hbm, out_hbm, idx_v, rows_v, a_v, h_v,
                   sem):
    wid = lax.axis_index("s") * SC_NC + lax.axis_index("c")
    node0 = wid * SC_NPW

    def chunk_body(ci, carry):
        base = node0 + ci * SC_CH
        pltpu.sync_copy(idx_hbm.at[pl.ds(base * K, SC_CH * K)], idx_v)
        pltpu.async_copy(bp_hbm.at[idx_v], rows_v, sem).wait()
        pltpu.sync_copy(a_hbm.at[pl.ds(base, SC_CH)], a_v)

        def node_body(n, carry2):
            for g in range(SC_G):
                z = a_v[n, pl.ds(g * SC_L, SC_L)]
                acc = None
                for j in range(K):
                    m = z + rows_v[n * K + j, pl.ds(g * SC_L, SC_L)]
                    m = jnp.maximum(m, NEG_SLOPE * m)
                    acc = m if acc is None else jnp.maximum(acc, m)
                h_v[n, pl.ds(g * SC_L, SC_L)] = acc
            return carry2

        lax.fori_loop(0, SC_CH, node_body, 0, unroll=False)
        pltpu.sync_copy(h_v, out_hbm.at[pl.ds(base, SC_CH)])
        return carry

    lax.fori_loop(0, SC_NCHUNK, chunk_body, 0, unroll=False)


def sc_gather_max(a_pad, bp_pad, idx_flat):
    mesh = plsc.VectorSubcoreMesh(core_axis_name="c", subcore_axis_name="s")
    kern = functools.partial(
        pl.kernel,
        mesh=mesh,
        out_type=jax.ShapeDtypeStruct((NPAD, C), jnp.float32),
        scratch_types=[
            pltpu.VMEM((SC_CH * K,), jnp.int32),
            pltpu.VMEM((SC_CH * K, C), jnp.float32),
            pltpu.VMEM((SC_CH, C), jnp.float32),
            pltpu.VMEM((SC_CH, C), jnp.float32),
            pltpu.SemaphoreType.DMA,
        ],
    )(_sc_gather_max)
    return kern(a_pad, bp_pad, idx_flat)


CB_RB = 2560


def _combine_body(x_ref, h00_ref, h01_ref, h10_ref, h11_ref, s0_ref, s1_ref,
                  s2_ref, out_ref):
    x = x_ref[...]
    g = None
    for ha_ref, hb_ref in ((h00_ref, h01_ref), (h10_ref, h11_ref)):
        x3 = jnp.concatenate([x, ha_ref[...], hb_ref[...]], axis=1)
        gb = None
        for s_ref in (s0_ref, s1_ref, s2_ref):
            sel = jnp.dot(x3, s_ref[...], preferred_element_type=jnp.float32,
                          precision=lax.Precision.HIGHEST)
            gb = sel if gb is None else jnp.maximum(gb, sel)
        g = gb if g is None else jnp.maximum(g, gb)
    out_ref[...] = g + x


def combine(x_pad, h00, h01, h10, h11, s0, s1, s2):
    return pl.pallas_call(
        _combine_body,
        grid=(NPAD // CB_RB,),
        in_specs=[pl.BlockSpec((CB_RB, C), lambda i: (i, 0))] * 5
        + [pl.BlockSpec((3 * C, C), lambda i: (0, 0))] * 3,
        out_specs=pl.BlockSpec((CB_RB, C), lambda i: (i, 0)),
        out_shape=jax.ShapeDtypeStruct((NPAD, C), jnp.float32),
    )(x_pad, h00, h01, h10, h11, s0, s1, s2)


def kernel(x, W0_0, b0_0, W0_1, b0_1, W1_0, b1_0, W1_1, b1_1):
    params = [
        [(W0_0, b0_0), (W0_1, b0_1)],
        [(W1_0, b1_0), (W1_1, b1_1)],
    ]
    x_pad = jnp.pad(x, ((0, NPAD - N), (0, 0)))

    idx32 = knn_top32(x_pad)

    hs = [[None, None], [None, None]]
    for b in range(2):
        nbr = idx32[:, :K] if b == 0 else idx32[:, 0 : 2 * K : 2]
        idx_flat = nbr.reshape(-1)
        xc = x_pad
        for blk in range(2):
            W, bias = params[b][blk]
            in_ch = xc.shape[1]
            wcat = jnp.concatenate([W[:in_ch] - W[in_ch:], W[in_ch:]], axis=1)
            bcat = jnp.concatenate([bias, jnp.zeros((C,), jnp.float32)])[None, :]
            ab = node_matmul(xc, wcat, bcat)
            a_pad, bp_pad = ab[:, :C], ab[:, C:]
            h = sc_gather_max(a_pad, bp_pad, idx_flat)
            hs[b][blk] = h
            xc = jnp.concatenate([xc, h], axis=-1)

    r = jnp.arange(3 * C)[:, None]
    c3 = 3 * jnp.arange(C)[None, :]
    sels = [(r == c3 + t).astype(jnp.float32) for t in range(3)]
    out = combine(x_pad, hs[0][0], hs[0][1], hs[1][0], hs[1][1], *sels)
    return out[:N]


# speedup vs baseline: 1.1414x; 1.0491x over previous
import functools

import jax
import jax.numpy as jnp
from jax import lax
from jax.experimental import pallas as pl
from jax.experimental.pallas import tpu as pltpu
from jax.experimental.pallas import tpu_sc as plsc

N = 10000
C = 128
K = 16
NPAD = 10240
NEG_SLOPE = 0.2


KNN_RB = 512
KNN_T = 2 * K


BIGF = 3.0e38


def _knn_body(xb_ref, xf_ref, sqb_ref, sqr_ref, out_ref, d_ref,
              m_ref, acc_ref):
    t = pl.program_id(1)

    @pl.when(t == 0)
    def _init():
        xb = xb_ref[...]
        xf = xf_ref[...]
        d2 = lax.dot_general(xb, xf, (((1,), (1,)), ((), ())),
                             preferred_element_type=jnp.float32)
        d = sqb_ref[...] - 2.0 * d2 + sqr_ref[...]
        rb = pl.program_id(0)
        cols = lax.broadcasted_iota(jnp.int32, (KNN_RB, NPAD), 1)
        rows = lax.broadcasted_iota(jnp.int32, (KNN_RB, NPAD), 0) + rb * KNN_RB
        d = jnp.where((cols == rows) | (cols >= N), BIGF, d)
        d_ref[...] = d
        m_ref[...] = jnp.min(d, axis=1, keepdims=True)
        acc_ref[...] = jnp.zeros((KNN_RB, KNN_T), jnp.int32)

    @pl.when(t > 0)
    def _extract():
        d = d_ref[...]
        m = m_ref[...]
        iota_b = lax.broadcasted_iota(jnp.int32, (KNN_RB, NPAD), 1
                                      ).astype(jnp.float32)
        cand = jnp.where(d == m, iota_b, BIGF)
        jf = jnp.min(cand, axis=1, keepdims=True)
        lane = lax.broadcasted_iota(jnp.int32, (KNN_RB, KNN_T), 1)
        acc_ref[...] = jnp.where(lane == t - 1, jf.astype(jnp.int32),
                                 acc_ref[...])

        @pl.when(t < KNN_T)
        def _mask_and_next_min():
            iota_a = lax.broadcasted_iota(jnp.int32, (KNN_RB, NPAD), 1)
            dn = jnp.where(iota_a == jf.astype(jnp.int32), BIGF, d)
            d_ref[...] = dn
            m_ref[...] = jnp.min(dn, axis=1, keepdims=True)

        @pl.when(t == KNN_T)
        def _emit():
            out_ref[...] = acc_ref[...]


def knn_top32(x_pad):
    sq = jnp.sum(x_pad * x_pad, axis=1)
    grid = (NPAD // KNN_RB, KNN_T + 1)
    return pl.pallas_call(
        _knn_body,
        grid=grid,
        in_specs=[
            pl.BlockSpec((KNN_RB, C), lambda rb, t: (rb, 0)),
            pl.BlockSpec((NPAD, C), lambda rb, t: (0, 0)),
            pl.BlockSpec((KNN_RB, 1), lambda rb, t: (rb, 0)),
            pl.BlockSpec((1, NPAD), lambda rb, t: (0, 0)),
        ],
        out_specs=pl.BlockSpec((KNN_RB, KNN_T), lambda rb, t: (rb, 0)),
        out_shape=jax.ShapeDtypeStruct((NPAD, KNN_T), jnp.int32),
        scratch_shapes=[
            pltpu.VMEM((KNN_RB, NPAD), jnp.float32),
            pltpu.VMEM((KNN_RB, 1), jnp.float32),
            pltpu.VMEM((KNN_RB, KNN_T), jnp.int32),
        ],
    )(x_pad, x_pad, sq[:, None], sq[None, :])


MM_RB = 1280


def _mm_body(x_ref, w_ref, b_ref, out_ref):
    acc = jnp.dot(x_ref[...], w_ref[...], preferred_element_type=jnp.float32,
                  precision=lax.Precision.HIGHEST)
    out_ref[...] = acc + b_ref[...]


def node_matmul(xc, wcat, bcat):
    n, in_ch = xc.shape
    oc = wcat.shape[1]
    return pl.pallas_call(
        _mm_body,
        grid=(n // MM_RB,),
        in_specs=[
            pl.BlockSpec((MM_RB, in_ch), lambda i: (i, 0)),
            pl.BlockSpec((in_ch, oc), lambda i: (0, 0)),
            pl.BlockSpec((1, oc), lambda i: (0, 0)),
        ],
        out_specs=pl.BlockSpec((MM_RB, oc), lambda i: (i, 0)),
        out_shape=jax.ShapeDtypeStruct((n, oc), jnp.float32),
    )(xc, wcat, bcat)


SC_INFO = plsc.get_sparse_core_info()
SC_NC = SC_INFO.num_cores
SC_NS = SC_INFO.num_subcores
SC_NW = SC_NC * SC_NS
SC_NPW = NPAD // SC_NW
SC_CH = 8
SC_NCHUNK = SC_NPW // SC_CH
SC_L = 16
SC_G = C // SC_L


def _sc_gather_max(a_hbm, bp_hbm, idx_hbm, out_hbm, idx_v, rows_v, a_v, h_v,
                   sem):
    wid = lax.axis_index("s") * SC_NC + lax.axis_index("c")
    node0 = wid * SC_NPW

    def chunk_body(ci, carry):
        base = node0 + ci * SC_CH
        pltpu.sync_copy(idx_hbm.at[pl.ds(base * K, SC_CH * K)], idx_v)
        pltpu.async_copy(bp_hbm.at[idx_v], rows_v, sem).wait()
        pltpu.sync_copy(a_hbm.at[pl.ds(base, SC_CH)], a_v)

        def node_body(n, carry2):
            for g in range(SC_G):
                z = a_v[n, pl.ds(g * SC_L, SC_L)]
                acc = None
                for j in range(K):
                    m = z + rows_v[n * K + j, pl.ds(g * SC_L, SC_L)]
                    m = jnp.maximum(m, NEG_SLOPE * m)
                    acc = m if acc is None else jnp.maximum(acc, m)
                h_v[n, pl.ds(g * SC_L, SC_L)] = acc
            return carry2

        lax.fori_loop(0, SC_CH, node_body, 0, unroll=False)
        pltpu.sync_copy(h_v, out_hbm.at[pl.ds(base, SC_CH)])
        return carry

    lax.fori_loop(0, SC_NCHUNK, chunk_body, 0, unroll=False)


def sc_gather_max(a_pad, bp_pad, idx_flat):
    mesh = plsc.VectorSubcoreMesh(core_axis_name="c", subcore_axis_name="s")
    kern = functools.partial(
        pl.kernel,
        mesh=mesh,
        out_type=jax.ShapeDtypeStruct((NPAD, C), jnp.float32),
        scratch_types=[
            pltpu.VMEM((SC_CH * K,), jnp.int32),
            pltpu.VMEM((SC_CH * K, C), jnp.float32),
            pltpu.VMEM((SC_CH, C), jnp.float32),
            pltpu.VMEM((SC_CH, C), jnp.float32),
            pltpu.SemaphoreType.DMA,
        ],
    )(_sc_gather_max)
    return kern(a_pad, bp_pad, idx_flat)


CB_RB = 2560


def _combine_body(x_ref, h00_ref, h01_ref, h10_ref, h11_ref, s0_ref, s1_ref,
                  s2_ref, out_ref):
    x = x_ref[...]
    g = None
    for ha_ref, hb_ref in ((h00_ref, h01_ref), (h10_ref, h11_ref)):
        x3 = jnp.concatenate([x, ha_ref[...], hb_ref[...]], axis=1)
        gb = None
        for s_ref in (s0_ref, s1_ref, s2_ref):
            sel = jnp.dot(x3, s_ref[...], preferred_element_type=jnp.float32,
                          precision=lax.Precision.HIGHEST)
            gb = sel if gb is None else jnp.maximum(gb, sel)
        g = gb if g is None else jnp.maximum(g, gb)
    out_ref[...] = g + x


def combine(x_pad, h00, h01, h10, h11, s0, s1, s2):
    return pl.pallas_call(
        _combine_body,
        grid=(NPAD // CB_RB,),
        in_specs=[pl.BlockSpec((CB_RB, C), lambda i: (i, 0))] * 5
        + [pl.BlockSpec((3 * C, C), lambda i: (0, 0))] * 3,
        out_specs=pl.BlockSpec((CB_RB, C), lambda i: (i, 0)),
        out_shape=jax.ShapeDtypeStruct((NPAD, C), jnp.float32),
    )(x_pad, h00, h01, h10, h11, s0, s1, s2)


def kernel(x, W0_0, b0_0, W0_1, b0_1, W1_0, b1_0, W1_1, b1_1):
    params = [
        [(W0_0, b0_0), (W0_1, b0_1)],
        [(W1_0, b1_0), (W1_1, b1_1)],
    ]
    x_pad = jnp.pad(x, ((0, NPAD - N), (0, 0)))

    idx32 = knn_top32(x_pad)

    hs = [[None, None], [None, None]]
    for b in range(2):
        nbr = idx32[:, :K] if b == 0 else idx32[:, 0 : 2 * K : 2]
        idx_flat = nbr.reshape(-1)
        xc = x_pad
        for blk in range(2):
            W, bias = params[b][blk]
            in_ch = xc.shape[1]
            wcat = jnp.concatenate([W[:in_ch] - W[in_ch:], W[in_ch:]], axis=1)
            bcat = jnp.concatenate([bias, jnp.zeros((C,), jnp.float32)])[None, :]
            ab = node_matmul(xc, wcat, bcat)
            a_pad, bp_pad = ab[:, :C], ab[:, C:]
            h = sc_gather_max(a_pad, bp_pad, idx_flat)
            hs[b][blk] = h
            xc = jnp.concatenate([xc, h], axis=-1)

    r = jnp.arange(3 * C)[:, None]
    c3 = 3 * jnp.arange(C)[None, :]
    sels = [(r == c3 + t).astype(jnp.float32) for t in range(3)]
    out = combine(x_pad, hs[0][0], hs[0][1], hs[1][0], hs[1][1], *sels)
    return out[:N]


# speedup vs baseline: 1.1814x; 1.0350x over previous
import functools

import jax
import jax.numpy as jnp
from jax import lax
from jax.experimental import pallas as pl
from jax.experimental.pallas import tpu as pltpu
from jax.experimental.pallas import tpu_sc as plsc

N = 10000
C = 128
K = 16
NPAD = 10240
NEG_SLOPE = 0.2


KNN_RB = 512
KNN_T = 2 * K


BIGF = 3.0e38


def _knn_body(xb_ref, xf_ref, sqb_ref, sqr_ref, out_ref, d_ref,
              m_ref, acc_ref):
    t = pl.program_id(1)

    @pl.when(t == 0)
    def _init():
        xb = xb_ref[...]
        xf = xf_ref[...]
        d2 = lax.dot_general(xb, xf, (((1,), (1,)), ((), ())),
                             preferred_element_type=jnp.float32)
        d = sqb_ref[...] - 2.0 * d2 + sqr_ref[...]
        rb = pl.program_id(0)
        cols = lax.broadcasted_iota(jnp.int32, (KNN_RB, NPAD), 1)
        rows = lax.broadcasted_iota(jnp.int32, (KNN_RB, NPAD), 0) + rb * KNN_RB
        d = jnp.where((cols == rows) | (cols >= N), BIGF, d)
        d_ref[...] = d
        m_ref[...] = jnp.min(d, axis=1, keepdims=True)
        acc_ref[...] = jnp.zeros((KNN_RB, KNN_T), jnp.int32)

    @pl.when(t > 0)
    def _extract():
        d = d_ref[...]
        m = m_ref[...]
        iota_b = lax.broadcasted_iota(jnp.int32, (KNN_RB, NPAD), 1
                                      ).astype(jnp.float32)
        cand = jnp.where(d == m, iota_b, BIGF)
        jf = jnp.min(cand, axis=1, keepdims=True)
        lane = lax.broadcasted_iota(jnp.int32, (KNN_RB, KNN_T), 1)
        acc_ref[...] = jnp.where(lane == t - 1, jf.astype(jnp.int32),
                                 acc_ref[...])

        @pl.when(t < KNN_T)
        def _mask_and_next_min():
            iota_a = lax.broadcasted_iota(jnp.int32, (KNN_RB, NPAD), 1)
            dn = jnp.where(iota_a == jf.astype(jnp.int32), BIGF, d)
            d_ref[...] = dn
            m_ref[...] = jnp.min(dn, axis=1, keepdims=True)

        @pl.when(t == KNN_T)
        def _emit():
            out_ref[...] = acc_ref[...]


def knn_top32(x_pad):
    sq = jnp.sum(x_pad * x_pad, axis=1)
    grid = (NPAD // KNN_RB, KNN_T + 1)
    return pl.pallas_call(
        _knn_body,
        grid=grid,
        in_specs=[
            pl.BlockSpec((KNN_RB, C), lambda rb, t: (rb, 0)),
            pl.BlockSpec((NPAD, C), lambda rb, t: (0, 0)),
            pl.BlockSpec((KNN_RB, 1), lambda rb, t: (rb, 0)),
            pl.BlockSpec((1, NPAD), lambda rb, t: (0, 0)),
        ],
        out_specs=pl.BlockSpec((KNN_RB, KNN_T), lambda rb, t: (rb, 0)),
        out_shape=jax.ShapeDtypeStruct((NPAD, KNN_T), jnp.int32),
        scratch_shapes=[
            pltpu.VMEM((KNN_RB, NPAD), jnp.float32),
            pltpu.VMEM((KNN_RB, 1), jnp.float32),
            pltpu.VMEM((KNN_RB, KNN_T), jnp.int32),
        ],
    )(x_pad, x_pad, sq[:, None], sq[None, :])


MM_RB = 1280


def _mm_body(x_ref, w_ref, b_ref, out_ref):
    acc = jnp.dot(x_ref[...], w_ref[...], preferred_element_type=jnp.float32,
                  precision=lax.Precision.HIGHEST)
    out_ref[...] = acc + b_ref[...]


def node_matmul(xc, wcat, bcat):
    n, in_ch = xc.shape
    oc = wcat.shape[1]
    return pl.pallas_call(
        _mm_body,
        grid=(n // MM_RB,),
        in_specs=[
            pl.BlockSpec((MM_RB, in_ch), lambda i: (i, 0)),
            pl.BlockSpec((in_ch, oc), lambda i: (0, 0)),
            pl.BlockSpec((1, oc), lambda i: (0, 0)),
        ],
        out_specs=pl.BlockSpec((MM_RB, oc), lambda i: (i, 0)),
        out_shape=jax.ShapeDtypeStruct((n, oc), jnp.float32),
    )(xc, wcat, bcat)


SC_INFO = plsc.get_sparse_core_info()
SC_NC = SC_INFO.num_cores
SC_NS = SC_INFO.num_subcores
SC_NW = SC_NC * SC_NS
SC_NPW = NPAD // SC_NW
SC_CH = 32
SC_NGATH = SC_CH * K // 128
SC_NCHUNK = SC_NPW // SC_CH
SC_L = 16
SC_G = C // SC_L


def _sc_gather_max(a_hbm, bp_hbm, idx_hbm, out_hbm, idx_v, rows_v, a_v, h_v,
                   sem):
    wid = lax.axis_index("s") * SC_NC + lax.axis_index("c")
    node0 = wid * SC_NPW

    def chunk_body(ci, carry):
        base = node0 + ci * SC_CH
        pltpu.sync_copy(idx_hbm.at[pl.ds(base * K, SC_CH * K)], idx_v)
        handles = []
        for g in range(SC_NGATH):
            handles.append(pltpu.async_copy(
                bp_hbm.at[idx_v.at[pl.ds(g * 128, 128)]],
                rows_v.at[pl.ds(g * 128, 128)], sem))
        pltpu.sync_copy(a_hbm.at[pl.ds(base, SC_CH)], a_v)
        for h in handles:
            h.wait()

        def node_body(n, carry2):
            for g in range(SC_G):
                z = a_v[n, pl.ds(g * SC_L, SC_L)]
                acc = None
                for j in range(K):
                    m = z + rows_v[n * K + j, pl.ds(g * SC_L, SC_L)]
                    m = jnp.maximum(m, NEG_SLOPE * m)
                    acc = m if acc is None else jnp.maximum(acc, m)
                h_v[n, pl.ds(g * SC_L, SC_L)] = acc
            return carry2

        lax.fori_loop(0, SC_CH, node_body, 0, unroll=False)
        pltpu.sync_copy(h_v, out_hbm.at[pl.ds(base, SC_CH)])
        return carry

    lax.fori_loop(0, SC_NCHUNK, chunk_body, 0, unroll=False)


def sc_gather_max(a_pad, bp_pad, idx_flat):
    mesh = plsc.VectorSubcoreMesh(core_axis_name="c", subcore_axis_name="s")
    kern = functools.partial(
        pl.kernel,
        mesh=mesh,
        out_type=jax.ShapeDtypeStruct((NPAD, C), jnp.float32),
        scratch_types=[
            pltpu.VMEM((SC_CH * K,), jnp.int32),
            pltpu.VMEM((SC_CH * K, C), jnp.float32),
            pltpu.VMEM((SC_CH, C), jnp.float32),
            pltpu.VMEM((SC_CH, C), jnp.float32),
            pltpu.SemaphoreType.DMA,
        ],
    )(_sc_gather_max)
    return kern(a_pad, bp_pad, idx_flat)


CB_RB = 2560


def _combine_body(x_ref, h00_ref, h01_ref, h10_ref, h11_ref, s0_ref, s1_ref,
                  s2_ref, out_ref):
    x = x_ref[...]
    g = None
    for ha_ref, hb_ref in ((h00_ref, h01_ref), (h10_ref, h11_ref)):
        x3 = jnp.concatenate([x, ha_ref[...], hb_ref[...]], axis=1)
        gb = None
        for s_ref in (s0_ref, s1_ref, s2_ref):
            sel = jnp.dot(x3, s_ref[...], preferred_element_type=jnp.float32,
                          precision=lax.Precision.HIGHEST)
            gb = sel if gb is None else jnp.maximum(gb, sel)
        g = gb if g is None else jnp.maximum(g, gb)
    out_ref[...] = g + x


def combine(x_pad, h00, h01, h10, h11, s0, s1, s2):
    return pl.pallas_call(
        _combine_body,
        grid=(NPAD // CB_RB,),
        in_specs=[pl.BlockSpec((CB_RB, C), lambda i: (i, 0))] * 5
        + [pl.BlockSpec((3 * C, C), lambda i: (0, 0))] * 3,
        out_specs=pl.BlockSpec((CB_RB, C), lambda i: (i, 0)),
        out_shape=jax.ShapeDtypeStruct((NPAD, C), jnp.float32),
    )(x_pad, h00, h01, h10, h11, s0, s1, s2)


def kernel(x, W0_0, b0_0, W0_1, b0_1, W1_0, b1_0, W1_1, b1_1):
    params = [
        [(W0_0, b0_0), (W0_1, b0_1)],
        [(W1_0, b1_0), (W1_1, b1_1)],
    ]
    x_pad = jnp.pad(x, ((0, NPAD - N), (0, 0)))

    idx32 = knn_top32(x_pad)

    hs = [[None, None], [None, None]]
    for b in range(2):
        nbr = idx32[:, :K] if b == 0 else idx32[:, 0 : 2 * K : 2]
        idx_flat = nbr.reshape(-1)
        xc = x_pad
        for blk in range(2):
            W, bias = params[b][blk]
            in_ch = xc.shape[1]
            wcat = jnp.concatenate([W[:in_ch] - W[in_ch:], W[in_ch:]], axis=1)
            bcat = jnp.concatenate([bias, jnp.zeros((C,), jnp.float32)])[None, :]
            ab = node_matmul(xc, wcat, bcat)
            a_pad, bp_pad = ab[:, :C], ab[:, C:]
            h = sc_gather_max(a_pad, bp_pad, idx_flat)
            hs[b][blk] = h
            xc = jnp.concatenate([xc, h], axis=-1)

    r = jnp.arange(3 * C)[:, None]
    c3 = 3 * jnp.arange(C)[None, :]
    sels = [(r == c3 + t).astype(jnp.float32) for t in range(3)]
    out = combine(x_pad, hs[0][0], hs[0][1], hs[1][0], hs[1][1], *sels)
    return out[:N]


# speedup vs baseline: 1.1911x; 1.0083x over previous
import functools

import jax
import jax.numpy as jnp
from jax import lax
from jax.experimental import pallas as pl
from jax.experimental.pallas import tpu as pltpu
from jax.experimental.pallas import tpu_sc as plsc

N = 10000
C = 128
K = 16
NPAD = 10240
NEG_SLOPE = 0.2


KNN_RB = 512
KNN_T = 2 * K


BIGF = 3.0e38


def _knn_body(xb_ref, xf_ref, sqb_ref, sqr_ref, out_ref, d_ref,
              m_ref, acc_ref):
    t = pl.program_id(1)

    @pl.when(t == 0)
    def _init():
        xb = xb_ref[...]
        xf = xf_ref[...]
        d2 = lax.dot_general(xb, xf, (((1,), (1,)), ((), ())),
                             preferred_element_type=jnp.float32)
        d = sqb_ref[...] - 2.0 * d2 + sqr_ref[...]
        rb = pl.program_id(0)
        cols = lax.broadcasted_iota(jnp.int32, (KNN_RB, NPAD), 1)
        rows = lax.broadcasted_iota(jnp.int32, (KNN_RB, NPAD), 0) + rb * KNN_RB
        d = jnp.where((cols == rows) | (cols >= N), BIGF, d)
        d_ref[...] = d
        m_ref[...] = jnp.min(d, axis=1, keepdims=True)
        acc_ref[...] = jnp.zeros((KNN_RB, KNN_T), jnp.int32)

    @pl.when(t > 0)
    def _extract():
        d = d_ref[...]
        m = m_ref[...]
        iota_b = lax.broadcasted_iota(jnp.int32, (KNN_RB, NPAD), 1
                                      ).astype(jnp.float32)
        cand = jnp.where(d == m, iota_b, BIGF)
        jf = jnp.min(cand, axis=1, keepdims=True)
        lane = lax.broadcasted_iota(jnp.int32, (KNN_RB, KNN_T), 1)
        acc_ref[...] = jnp.where(lane == t - 1, jf.astype(jnp.int32),
                                 acc_ref[...])

        @pl.when(t < KNN_T)
        def _mask_and_next_min():
            iota_a = lax.broadcasted_iota(jnp.int32, (KNN_RB, NPAD), 1)
            dn = jnp.where(iota_a == jf.astype(jnp.int32), BIGF, d)
            d_ref[...] = dn
            m_ref[...] = jnp.min(dn, axis=1, keepdims=True)

        @pl.when(t == KNN_T)
        def _emit():
            out_ref[...] = acc_ref[...]


def knn_top32(x_pad):
    sq = jnp.sum(x_pad * x_pad, axis=1)
    grid = (NPAD // KNN_RB, KNN_T + 1)
    return pl.pallas_call(
        _knn_body,
        grid=grid,
        in_specs=[
            pl.BlockSpec((KNN_RB, C), lambda rb, t: (rb, 0)),
            pl.BlockSpec((NPAD, C), lambda rb, t: (0, 0)),
            pl.BlockSpec((KNN_RB, 1), lambda rb, t: (rb, 0)),
            pl.BlockSpec((1, NPAD), lambda rb, t: (0, 0)),
        ],
        out_specs=pl.BlockSpec((KNN_RB, KNN_T), lambda rb, t: (rb, 0)),
        out_shape=jax.ShapeDtypeStruct((NPAD, KNN_T), jnp.int32),
        scratch_shapes=[
            pltpu.VMEM((KNN_RB, NPAD), jnp.float32),
            pltpu.VMEM((KNN_RB, 1), jnp.float32),
            pltpu.VMEM((KNN_RB, KNN_T), jnp.int32),
        ],
    )(x_pad, x_pad, sq[:, None], sq[None, :])


MM_RB = 1280


def _mm_body(x_ref, w_ref, b_ref, a_ref, bp_ref):
    acc = jnp.dot(x_ref[...], w_ref[...], preferred_element_type=jnp.float32,
                  precision=lax.Precision.HIGHEST)
    a_ref[...] = acc[:, :C] + b_ref[...]
    bp_ref[...] = acc[:, C:]


def node_matmul(xc, wcat, bias):
    n, in_ch = xc.shape
    oc = wcat.shape[1]
    return pl.pallas_call(
        _mm_body,
        grid=(n // MM_RB,),
        in_specs=[
            pl.BlockSpec((MM_RB, in_ch), lambda i: (i, 0)),
            pl.BlockSpec((in_ch, oc), lambda i: (0, 0)),
            pl.BlockSpec((1, C), lambda i: (0, 0)),
        ],
        out_specs=[
            pl.BlockSpec((MM_RB, C), lambda i: (i, 0)),
            pl.BlockSpec((MM_RB, C), lambda i: (i, 0)),
        ],
        out_shape=[
            jax.ShapeDtypeStruct((n, C), jnp.float32),
            jax.ShapeDtypeStruct((n, C), jnp.float32),
        ],
    )(xc, wcat, bias)


SC_INFO = plsc.get_sparse_core_info()
SC_NC = SC_INFO.num_cores
SC_NS = SC_INFO.num_subcores
SC_NW = SC_NC * SC_NS
SC_NPW = NPAD // SC_NW
SC_CH = 32
SC_NGATH = SC_CH * K // 128
SC_NCHUNK = SC_NPW // SC_CH
SC_L = 16
SC_G = C // SC_L


def _sc_gather_max(a_hbm, bp_hbm, idx_hbm, out_hbm, idx_v, rows_v, a_v, h_v,
                   sem):
    wid = lax.axis_index("s") * SC_NC + lax.axis_index("c")
    node0 = wid * SC_NPW

    def chunk_body(ci, carry):
        base = node0 + ci * SC_CH
        pltpu.sync_copy(idx_hbm.at[pl.ds(base * K, SC_CH * K)], idx_v)
        handles = []
        for g in range(SC_NGATH):
            handles.append(pltpu.async_copy(
                bp_hbm.at[idx_v.at[pl.ds(g * 128, 128)]],
                rows_v.at[pl.ds(g * 128, 128)], sem))
        pltpu.sync_copy(a_hbm.at[pl.ds(base, SC_CH)], a_v)
        for h in handles:
            h.wait()

        def node_body(n, carry2):
            for g in range(SC_G):
                z = a_v[n, pl.ds(g * SC_L, SC_L)]
                acc = None
                for j in range(K):
                    m = z + rows_v[n * K + j, pl.ds(g * SC_L, SC_L)]
                    m = jnp.maximum(m, NEG_SLOPE * m)
                    acc = m if acc is None else jnp.maximum(acc, m)
                h_v[n, pl.ds(g * SC_L, SC_L)] = acc
            return carry2

        lax.fori_loop(0, SC_CH, node_body, 0, unroll=False)
        pltpu.sync_copy(h_v, out_hbm.at[pl.ds(base, SC_CH)])
        return carry

    lax.fori_loop(0, SC_NCHUNK, chunk_body, 0, unroll=False)


def sc_gather_max(a_pad, bp_pad, idx_flat):
    mesh = plsc.VectorSubcoreMesh(core_axis_name="c", subcore_axis_name="s")
    kern = functools.partial(
        pl.kernel,
        mesh=mesh,
        out_type=jax.ShapeDtypeStruct((NPAD, C), jnp.float32),
        scratch_types=[
            pltpu.VMEM((SC_CH * K,), jnp.int32),
            pltpu.VMEM((SC_CH * K, C), jnp.float32),
            pltpu.VMEM((SC_CH, C), jnp.float32),
            pltpu.VMEM((SC_CH, C), jnp.float32),
            pltpu.SemaphoreType.DMA,
        ],
    )(_sc_gather_max)
    return kern(a_pad, bp_pad, idx_flat)


CB_RB = 2560


def _combine_body(x_ref, h00_ref, h01_ref, h10_ref, h11_ref, s0_ref, s1_ref,
                  s2_ref, out_ref):
    x = x_ref[...]
    g = None
    for ha_ref, hb_ref in ((h00_ref, h01_ref), (h10_ref, h11_ref)):
        x3 = jnp.concatenate([x, ha_ref[...], hb_ref[...]], axis=1)
        gb = None
        for s_ref in (s0_ref, s1_ref, s2_ref):
            sel = jnp.dot(x3, s_ref[...], preferred_element_type=jnp.float32,
                          precision=lax.Precision.HIGHEST)
            gb = sel if gb is None else jnp.maximum(gb, sel)
        g = gb if g is None else jnp.maximum(g, gb)
    out_ref[...] = g + x


def combine(x_pad, h00, h01, h10, h11, s0, s1, s2):
    return pl.pallas_call(
        _combine_body,
        grid=(NPAD // CB_RB,),
        in_specs=[pl.BlockSpec((CB_RB, C), lambda i: (i, 0))] * 5
        + [pl.BlockSpec((3 * C, C), lambda i: (0, 0))] * 3,
        out_specs=pl.BlockSpec((CB_RB, C), lambda i: (i, 0)),
        out_shape=jax.ShapeDtypeStruct((NPAD, C), jnp.float32),
    )(x_pad, h00, h01, h10, h11, s0, s1, s2)


def kernel(x, W0_0, b0_0, W0_1, b0_1, W1_0, b1_0, W1_1, b1_1):
    params = [
        [(W0_0, b0_0), (W0_1, b0_1)],
        [(W1_0, b1_0), (W1_1, b1_1)],
    ]
    x_pad = jnp.pad(x, ((0, NPAD - N), (0, 0)))

    idx32 = knn_top32(x_pad)

    hs = [[None, None], [None, None]]
    for b in range(2):
        nbr = idx32[:, :K] if b == 0 else idx32[:, 0 : 2 * K : 2]
        idx_flat = nbr.reshape(-1)
        xc = x_pad
        for blk in range(2):
            W, bias = params[b][blk]
            in_ch = xc.shape[1]
            wcat = jnp.concatenate([W[:in_ch] - W[in_ch:], W[in_ch:]], axis=1)
            a_pad, bp_pad = node_matmul(xc, wcat, bias[None, :])
            h = sc_gather_max(a_pad, bp_pad, idx_flat)
            hs[b][blk] = h
            xc = jnp.concatenate([xc, h], axis=-1)

    r = jnp.arange(3 * C)[:, None]
    c3 = 3 * jnp.arange(C)[None, :]
    sels = [(r == c3 + t).astype(jnp.float32) for t in range(3)]
    out = combine(x_pad, hs[0][0], hs[0][1], hs[1][0], hs[1][1], *sels)
    return out[:N]
